# Initial kernel scaffold; baseline (speedup 1.0000x reference)
import jax
import jax.numpy as jnp
from jax.experimental import pallas as pl


def kernel(S, mu, A):
    raise NotImplementedError("write your pallas kernel here")


# speedup vs baseline: 104.8426x; 104.8426x over previous
import functools

import jax
import jax.numpy as jnp
from jax import lax
from jax.experimental import pallas as pl
from jax.experimental.pallas import tpu as pltpu
from jax.experimental.pallas import tpu_sc as plsc

NC = 2
NS = 16
LANES = 16
NW = NC * NS

VPAD = 1024
CHUNK_ROWS = 8
NBUF = 2


def _build_counts_kernel(B, HPAD):
    rows_per_w = B // NW
    n_chunks = rows_per_w // CHUNK_ROWS
    mesh = plsc.VectorSubcoreMesh(
        core_axis_name="c", subcore_axis_name="s",
        num_cores=NC, num_subcores=NS)

    @functools.partial(
        pl.kernel,
        out_type=jax.ShapeDtypeStruct((B * VPAD,), jnp.float32),
        mesh=mesh,
        scratch_types=[
            pltpu.VMEM((rows_per_w, HPAD), jnp.int32),
            pltpu.VMEM((CHUNK_ROWS * VPAD,), jnp.float32),
            pltpu.VMEM((CHUNK_ROWS * VPAD,), jnp.float32),
            pltpu.SemaphoreType.DMA,
            pltpu.SemaphoreType.DMA,
        ],
        compiler_params=pltpu.CompilerParams(needs_layout_passes=False),
    )
    def build_counts(s_hbm, c_hbm, s_v, hist0, hist1, sem0, sem1):
        wid = lax.axis_index("s") * NC + lax.axis_index("c")
        base = wid * rows_per_w
        pltpu.sync_copy(s_hbm.at[pl.ds(base, rows_per_w)], s_v)
        sems = [sem0, sem1]
        hists = [hist0, hist1]

        zeros16 = jnp.zeros((LANES,), jnp.float32)
        ones16 = jnp.ones((LANES,), jnp.float32)

        def do_chunk(k, buf):
            def zero_body(i, _):
                hists[buf][pl.ds(i * LANES, LANES)] = zeros16
                return 0
            lax.fori_loop(0, CHUNK_ROWS * VPAD // LANES, zero_body, 0)
            for r in range(CHUNK_ROWS):
                row = k * CHUNK_ROWS + r
                for j in range(HPAD // LANES):
                    idx = s_v[row, pl.ds(j * LANES, LANES)]
                    plsc.addupdate_scatter(
                        hists[buf], [idx + r * VPAD], ones16)
            pltpu.async_copy(
                hists[buf],
                c_hbm.at[pl.ds((base + k * CHUNK_ROWS) * VPAD,
                               CHUNK_ROWS * VPAD)],
                sems[buf])

        def loop_body(k2, _):
            for buf in range(NBUF):
                k = k2 * NBUF + buf
                @pl.when(k2 > 0)
                def _wait():
                    pltpu.make_async_copy(
                        hists[buf],
                        c_hbm.at[pl.ds((base + (k - NBUF) * CHUNK_ROWS)
                                       * VPAD, CHUNK_ROWS * VPAD)],
                        sems[buf]).wait()
                do_chunk(k, buf)
            return 0

        lax.fori_loop(0, n_chunks // NBUF, loop_body, 0)
        for buf in range(NBUF):
            k = n_chunks - NBUF + buf
            pltpu.make_async_copy(
                hists[buf],
                c_hbm.at[pl.ds((base + k * CHUNK_ROWS) * VPAD,
                               CHUNK_ROWS * VPAD)],
                sems[buf]).wait()

    return build_counts


def _tc_body(hist_len, c_ref, a_ref, mu_ref, o_ref):
    c = c_ref[...]
    p = jnp.dot(c, a_ref[...], preferred_element_type=jnp.float32)
    v = jnp.sum(p * p, axis=1, keepdims=True)
    m = jnp.dot(c, mu_ref[...], preferred_element_type=jnp.float32)
    o_ref[...] = jax.nn.sigmoid(m * lax.rsqrt(hist_len + v))


def kernel(S, mu, A):
    B, H = S.shape
    D = A.shape[1]
    hpad = ((H + LANES - 1) // LANES) * LANES
    s_pad = jnp.concatenate(
        [S.astype(jnp.int32),
         jnp.full((B, hpad - H), D, jnp.int32)], axis=1)
    counts = _build_counts_kernel(B, hpad)(s_pad).reshape(B, VPAD)

    a_pad = jnp.zeros((VPAD, D), jnp.float32).at[:A.shape[0]].set(A)
    mu_pad = jnp.zeros((VPAD, 1), jnp.float32).at[:mu.shape[0], 0].set(mu)

    blk = 512
    out2d = pl.pallas_call(
        functools.partial(_tc_body, float(H)),
        grid=(B // blk,),
        in_specs=[
            pl.BlockSpec((blk, VPAD), lambda i: (i, 0)),
            pl.BlockSpec((VPAD, D), lambda i: (0, 0)),
            pl.BlockSpec((VPAD, 1), lambda i: (0, 0)),
        ],
        out_specs=pl.BlockSpec((blk, 1), lambda i: (i, 0)),
        out_shape=jax.ShapeDtypeStruct((B, 1), jnp.float32),
    )(counts, a_pad, mu_pad)
    return out2d[:, 0]


# speedup vs baseline: 138.7189x; 1.3231x over previous
import functools

import jax
import jax.numpy as jnp
from jax import lax
from jax.experimental import pallas as pl
from jax.experimental.pallas import tpu as pltpu
from jax.experimental.pallas import tpu_sc as plsc

NC = 2
NS = 16
LANES = 16
NW = NC * NS

VPAD = 1024
CHUNK_ROWS = 8
NBUF = 2


def _build_counts_kernel(B, HPAD):
    rows_per_w = B // NW
    n_chunks = rows_per_w // CHUNK_ROWS
    mesh = plsc.VectorSubcoreMesh(
        core_axis_name="c", subcore_axis_name="s",
        num_cores=NC, num_subcores=NS)

    @functools.partial(
        pl.kernel,
        out_type=jax.ShapeDtypeStruct((B * VPAD,), jnp.float32),
        mesh=mesh,
        scratch_types=[
            pltpu.VMEM((rows_per_w, HPAD), jnp.int32),
            pltpu.VMEM((CHUNK_ROWS * VPAD,), jnp.float32),
            pltpu.VMEM((CHUNK_ROWS * VPAD,), jnp.float32),
            pltpu.SemaphoreType.DMA,
            pltpu.SemaphoreType.DMA,
        ],
        compiler_params=pltpu.CompilerParams(needs_layout_passes=False),
    )
    def build_counts(s_hbm, c_hbm, s_v, hist0, hist1, sem0, sem1):
        wid = lax.axis_index("s") * NC + lax.axis_index("c")
        base = wid * rows_per_w
        pltpu.sync_copy(s_hbm.at[pl.ds(base, rows_per_w)], s_v)
        sems = [sem0, sem1]
        hists = [hist0, hist1]

        zeros16 = jnp.zeros((LANES,), jnp.float32)
        ones16 = jnp.ones((LANES,), jnp.float32)

        def ship(k, buf):
            pltpu.async_copy(
                hists[buf],
                c_hbm.at[pl.ds((base + k * CHUNK_ROWS) * VPAD,
                               CHUNK_ROWS * VPAD)],
                sems[buf])

        def loop_body(k2, _):
            @pl.when(k2 > 0)
            def _wait():
                for buf in range(NBUF):
                    k = k2 * NBUF + buf
                    pltpu.make_async_copy(
                        hists[buf],
                        c_hbm.at[pl.ds((base + (k - NBUF) * CHUNK_ROWS)
                                       * VPAD, CHUNK_ROWS * VPAD)],
                        sems[buf]).wait()
            for i in range(CHUNK_ROWS * VPAD // LANES):
                for buf in range(NBUF):
                    hists[buf][pl.ds(i * LANES, LANES)] = zeros16
            for r in range(CHUNK_ROWS):
                idxs = []
                for buf in range(NBUF):
                    row = (k2 * NBUF + buf) * CHUNK_ROWS + r
                    for j in range(HPAD // LANES):
                        idxs.append(
                            (buf, s_v[row, pl.ds(j * LANES, LANES)]))
                for buf, idx in idxs:
                    plsc.addupdate_scatter(
                        hists[buf], [idx + r * VPAD], ones16)
            for buf in range(NBUF):
                ship(k2 * NBUF + buf, buf)
            return 0

        lax.fori_loop(0, n_chunks // NBUF, loop_body, 0)
        for buf in range(NBUF):
            k = n_chunks - NBUF + buf
            pltpu.make_async_copy(
                hists[buf],
                c_hbm.at[pl.ds((base + k * CHUNK_ROWS) * VPAD,
                               CHUNK_ROWS * VPAD)],
                sems[buf]).wait()

    return build_counts


def _tc_body(hist_len, c_ref, a_ref, mu_ref, o_ref):
    c = c_ref[...]
    p = jnp.dot(c, a_ref[...], preferred_element_type=jnp.float32)
    v = jnp.sum(p * p, axis=1, keepdims=True)
    m = jnp.dot(c, mu_ref[...], preferred_element_type=jnp.float32)
    o_ref[...] = jax.nn.sigmoid(m * lax.rsqrt(hist_len + v))


def kernel(S, mu, A):
    B, H = S.shape
    D = A.shape[1]
    hpad = ((H + LANES - 1) // LANES) * LANES
    s_pad = jnp.concatenate(
        [S.astype(jnp.int32),
         jnp.full((B, hpad - H), D, jnp.int32)], axis=1)
    counts = _build_counts_kernel(B, hpad)(s_pad).reshape(B, VPAD)

    a_pad = jnp.zeros((VPAD, D), jnp.float32).at[:A.shape[0]].set(A)
    mu_pad = jnp.zeros((VPAD, 1), jnp.float32).at[:mu.shape[0], 0].set(mu)

    blk = 512
    out2d = pl.pallas_call(
        functools.partial(_tc_body, float(H)),
        grid=(B // blk,),
        in_specs=[
            pl.BlockSpec((blk, VPAD), lambda i: (i, 0)),
            pl.BlockSpec((VPAD, D), lambda i: (0, 0)),
            pl.BlockSpec((VPAD, 1), lambda i: (0, 0)),
        ],
        out_specs=pl.BlockSpec((blk, 1), lambda i: (i, 0)),
        out_shape=jax.ShapeDtypeStruct((B, 1), jnp.float32),
    )(counts, a_pad, mu_pad)
    return out2d[:, 0]
